# baseline (speedup 1.0000x reference)
import functools

import jax
import jax.numpy as jnp
from jax import lax
from jax.experimental import pallas as pl
from jax.experimental.pallas import tpu as pltpu
from jax.experimental.pallas import tpu_sc as plsc

NUM_USERS = 100000
NUM_ITEMS = 1000000
D = 64
B = 16384

NC = 2
NS = 16
NW = NC * NS
BPW = B // NW
CHUNK = 128
NCH = BPW // CHUNK


def _body(uidx_hbm, iidx_hbm, utab_hbm, itab_hbm, out_hbm,
          uidx_v, iidx_v, ukey_v, ikey_v, urows_v, irows_v, out_v,
          sem_u, sem_i):
    wid = lax.axis_index("s") * NC + lax.axis_index("c")
    base = wid * BPW

    pltpu.sync_copy(uidx_hbm.at[pl.ds(wid * NCH, NCH)], uidx_v)
    pltpu.sync_copy(iidx_hbm.at[pl.ds(wid * NCH, NCH)], iidx_v)

    def keys(s, carry):
        c = s // 8
        g = s % 8
        uidx = uidx_v[c, pl.ds(g * 16, 16)]
        iidx = iidx_v[c, pl.ds(g * 16, 16)]
        ukey_v[c, pl.ds(g * 16, 16)] = jnp.right_shift(uidx, 1)
        ikey_v[c, pl.ds(g * 16, 16)] = jnp.right_shift(iidx, 1)
        return carry

    lax.fori_loop(0, NCH * 8, keys, 0)

    lanes = lax.iota(jnp.int32, 16)
    zero = jnp.zeros((16,), jnp.float32)
    one = jnp.full((16,), 1, jnp.int32)

    for p in range(2):
        copies = []
        for cc in range(2):
            c = 2 * p + cc
            copies.append(pltpu.async_copy(
                utab_hbm.at[ukey_v.at[c]],
                urows_v.at[pl.ds(cc * CHUNK, CHUNK)], sem_u))
            copies.append(pltpu.async_copy(
                itab_hbm.at[ikey_v.at[c]],
                irows_v.at[pl.ds(cc * CHUNK, CHUNK)], sem_i))
        for cp in copies:
            cp.wait()

        def group(g, carry):
            row = g * 16 + lanes
            c = 2 * p + g // 8
            gg = g % 8
            uoff = jnp.left_shift(
                jnp.bitwise_and(uidx_v[c, pl.ds(gg * 16, 16)], one), 6)
            ioff = jnp.left_shift(
                jnp.bitwise_and(iidx_v[c, pl.ds(gg * 16, 16)], one), 6)
            acc = [zero, zero, zero, zero]
            for d in range(D):
                dvec = jnp.full((16,), d, jnp.int32)
                u = plsc.load_gather(urows_v, [row, uoff + dvec])
                v = plsc.load_gather(irows_v, [row, ioff + dvec])
                acc[d % 4] = acc[d % 4] + (
                    jnp.maximum(u, 0.0) * jnp.maximum(v, 0.0))
            out_v[pl.ds(p * 256 + g * 16, 16)] = (
                (acc[0] + acc[1]) + (acc[2] + acc[3]))
            return carry

        lax.fori_loop(0, 256 // 16, group, 0)

    pltpu.sync_copy(out_v, out_hbm.at[pl.ds(base, BPW)])


@functools.partial(jax.jit, static_argnums=())
def _run(uidx2d, iidx2d, utab2, itab2):
    mesh = plsc.VectorSubcoreMesh(core_axis_name="c", subcore_axis_name="s")
    k = pl.kernel(
        _body,
        mesh=mesh,
        out_type=jax.ShapeDtypeStruct((B,), jnp.float32),
        scratch_types=[
            pltpu.VMEM((NCH, CHUNK), jnp.int32),
            pltpu.VMEM((NCH, CHUNK), jnp.int32),
            pltpu.VMEM((NCH, CHUNK), jnp.int32),
            pltpu.VMEM((NCH, CHUNK), jnp.int32),
            pltpu.VMEM((2 * CHUNK, 2 * D), jnp.float32),
            pltpu.VMEM((2 * CHUNK, 2 * D), jnp.float32),
            pltpu.VMEM((BPW,), jnp.float32),
            pltpu.SemaphoreType.DMA,
            pltpu.SemaphoreType.DMA,
        ],
        compiler_params=pltpu.CompilerParams(
            needs_layout_passes=False, use_tc_tiling_on_sc=True),
    )
    return k(uidx2d, iidx2d, utab2, itab2)


def kernel(user_indices, item_indices, user_table, item_table):
    uidx2d = user_indices.astype(jnp.int32).reshape(NW * NCH, CHUNK)
    iidx2d = item_indices.astype(jnp.int32).reshape(NW * NCH, CHUNK)
    utab2 = user_table.reshape(NUM_USERS // 2, 2 * D)
    itab2 = item_table.reshape(NUM_ITEMS // 2, 2 * D)
    return _run(uidx2d, iidx2d, utab2, itab2)
